# baseline (speedup 1.0000x reference)
import functools

import jax
import jax.numpy as jnp
from jax import lax
from jax.experimental import pallas as pl
from jax.experimental.pallas import tpu as pltpu
from jax.experimental.pallas import tpu_sc as plsc

N_NODES = 10000
N_PAD = 10240
H = 128
NSEG = 64
NC = 2
NS = 16
CH = 128
RPS = N_PAD // NS
BR = 1280
DEGW = 128

_f32 = jnp.float32


def _sc_mesh():
    return plsc.VectorSubcoreMesh(core_axis_name="c", subcore_axis_name="s")


@functools.cache
def _build_agg(K):
    assert K % 4 == 0
    K2 = K // 2

    @functools.partial(
        pl.kernel,
        mesh=_sc_mesh(),
        out_type=jax.ShapeDtypeStruct((NC, N_PAD, H), _f32),
        scratch_types=[
            pltpu.VMEM((K2, CH), jnp.int32),
            pltpu.VMEM((K2, CH), jnp.int32),
            pltpu.VMEM((CH, H), _f32),
            pltpu.VMEM((CH, H), _f32),
            pltpu.VMEM_SHARED((N_PAD, H), _f32),
            pltpu.SemaphoreType.DMA,
            pltpu.SemaphoreType.DMA,
        ],
    )
    def agg(y_hbm, src_hbm, dst_hbm, z_hbm, out_hbm, src_v, dst_v, rows0,
            rows1, acc_sh, sem0, sem1):
        c = lax.axis_index("c")
        s = lax.axis_index("s")
        pltpu.sync_copy(z_hbm.at[pl.ds(s * RPS, RPS)],
                        acc_sh.at[pl.ds(s * RPS, RPS)])
        plsc.subcore_barrier()

        def half(h):
            pltpu.sync_copy(src_hbm.at[c, s, pl.ds(h * K2, K2)], src_v)
            pltpu.sync_copy(dst_hbm.at[c, s, pl.ds(h * K2, K2)], dst_v)
            pltpu.async_copy(y_hbm.at[src_v.at[0]], rows0, sem0)

            def body(jj, carry):
                g = jj * 2
                pltpu.make_async_copy(y_hbm.at[src_v.at[g]], rows0,
                                      sem0).wait()
                pltpu.async_copy(y_hbm.at[src_v.at[g + 1]], rows1, sem1)
                pltpu.sync_copy(rows0, acc_sh.at[dst_v.at[g]], add=True)
                pltpu.make_async_copy(y_hbm.at[src_v.at[g + 1]], rows1,
                                      sem1).wait()

                @pl.when(g + 2 < K2)
                def _next():
                    pltpu.async_copy(y_hbm.at[src_v.at[g + 2]], rows0, sem0)

                pltpu.sync_copy(rows1, acc_sh.at[dst_v.at[g + 1]], add=True)
                return carry

            lax.fori_loop(0, K2 // 2, body, 0)

        half(0)
        half(1)
        plsc.subcore_barrier()
        pltpu.sync_copy(acc_sh.at[pl.ds(s * RPS, RPS)],
                        out_hbm.at[c, pl.ds(s * RPS, RPS)])

    return agg


@functools.cache
def _build_deg(K, degw=DEGW):
    @functools.partial(
        pl.kernel,
        mesh=_sc_mesh(),
        out_type=jax.ShapeDtypeStruct((NC, N_PAD, degw), _f32),
        scratch_types=[
            pltpu.VMEM((K, CH), jnp.int32),
            pltpu.VMEM((CH, degw), _f32),
            pltpu.VMEM_SHARED((N_PAD, degw), _f32),
        ],
    )
    def deg(dst_hbm, ones_hbm, z_hbm, out_hbm, dst_v, ones_v, deg_sh):
        c = lax.axis_index("c")
        s = lax.axis_index("s")
        pltpu.sync_copy(dst_hbm.at[c, s], dst_v)
        pltpu.sync_copy(ones_hbm, ones_v)
        pltpu.sync_copy(z_hbm.at[pl.ds(s * RPS, RPS)],
                        deg_sh.at[pl.ds(s * RPS, RPS)])
        plsc.subcore_barrier()

        def body(j, carry):
            pltpu.sync_copy(ones_v, deg_sh.at[dst_v.at[j]], add=True)
            return carry

        lax.fori_loop(0, K, body, 0)
        plsc.subcore_barrier()
        pltpu.sync_copy(deg_sh.at[pl.ds(s * RPS, RPS)],
                        out_hbm.at[c, pl.ds(s * RPS, RPS)])

    return deg


def _dinv_of(deg_blk):
    return lax.rsqrt(deg_blk[0, :, 0:1] + deg_blk[1, :, 0:1] + 1.0)


def _prep_body(x_ref, w_ref, deg_ref, y_ref):
    dinv = _dinv_of(deg_ref[...])
    y_ref[...] = jnp.dot(x_ref[...], w_ref[...],
                         preferred_element_type=_f32) * dinv


def _tc_prep(xpad, w, deg2):
    grid = N_PAD // BR
    return pl.pallas_call(
        _prep_body,
        grid=(grid,),
        in_specs=[
            pl.BlockSpec((BR, H), lambda i: (i, 0)),
            pl.BlockSpec((H, H), lambda i: (0, 0)),
            pl.BlockSpec((NC, BR, DEGW), lambda i: (0, i, 0)),
        ],
        out_specs=pl.BlockSpec((BR, H), lambda i: (i, 0)),
        out_shape=jax.ShapeDtypeStruct((N_PAD, H), _f32),
    )(xpad, w, deg2)


def _mid_body(acc_ref, y_ref, deg_ref, b_ref, w_ref, x_ref, ynext_ref):
    dinv = _dinv_of(deg_ref[...])
    acc = acc_ref[0] + acc_ref[1] + y_ref[...]
    xk = jnp.maximum(dinv * acc + b_ref[...], 0.0)
    row = pl.program_id(0) * BR + lax.broadcasted_iota(jnp.int32, (BR, 1), 0)
    xk = jnp.where(row < N_NODES, xk, 0.0)
    x_ref[...] = xk
    ynext_ref[...] = jnp.dot(xk, w_ref[...],
                             preferred_element_type=_f32) * dinv


def _mid_last_body(acc_ref, y_ref, deg_ref, b_ref, x_ref):
    dinv = _dinv_of(deg_ref[...])
    acc = acc_ref[0] + acc_ref[1] + y_ref[...]
    xk = jnp.maximum(dinv * acc + b_ref[...], 0.0)
    row = pl.program_id(0) * BR + lax.broadcasted_iota(jnp.int32, (BR, 1), 0)
    x_ref[...] = jnp.where(row < N_NODES, xk, 0.0)


def _tc_mid(acc2, y, deg2, b, w_next):
    grid = N_PAD // BR
    specs = [
        pl.BlockSpec((NC, BR, H), lambda i: (0, i, 0)),
        pl.BlockSpec((BR, H), lambda i: (i, 0)),
        pl.BlockSpec((NC, BR, DEGW), lambda i: (0, i, 0)),
        pl.BlockSpec((1, H), lambda i: (0, 0)),
    ]
    if w_next is None:
        return pl.pallas_call(
            _mid_last_body,
            grid=(grid,),
            in_specs=specs,
            out_specs=pl.BlockSpec((BR, H), lambda i: (i, 0)),
            out_shape=jax.ShapeDtypeStruct((N_PAD, H), _f32),
        )(acc2, y, deg2, b.reshape(1, H))
    return pl.pallas_call(
        _mid_body,
        grid=(grid,),
        in_specs=specs + [pl.BlockSpec((H, H), lambda i: (0, 0))],
        out_specs=[pl.BlockSpec((BR, H), lambda i: (i, 0)),
                   pl.BlockSpec((BR, H), lambda i: (i, 0))],
        out_shape=[jax.ShapeDtypeStruct((N_PAD, H), _f32),
                   jax.ShapeDtypeStruct((N_PAD, H), _f32)],
    )(acc2, y, deg2, b.reshape(1, H), w_next)


def _pool_body(b3_ref, x1_ref, x2_ref, x3_ref, wl1_ref, bl1_ref, wl2_ref,
               bl2_ref, out_ref, sums, cnts):
    i = pl.program_id(0)

    @pl.when(i == 0)
    def _init():
        sums[...] = jnp.zeros_like(sums)
        cnts[...] = jnp.zeros_like(cnts)

    bvals = b3_ref[...].reshape(1, CH)
    seg = lax.broadcasted_iota(jnp.int32, (NSEG, CH), 0)
    oh = (seg == bvals).astype(_f32)
    xj = jnp.concatenate([x1_ref[...], x2_ref[...], x3_ref[...]], axis=1)
    sums[...] += jnp.dot(oh, xj, preferred_element_type=_f32)
    cnts[...] += oh

    @pl.when(i == N_PAD // CH - 1)
    def _fin():
        cnt = jnp.sum(cnts[...], axis=1, keepdims=True)
        pooled = sums[...] / jnp.maximum(cnt, 1.0)
        h = jnp.maximum(
            jnp.dot(pooled, wl1_ref[...], preferred_element_type=_f32)
            + bl1_ref[...], 0.0)
        o = (jnp.dot(h, wl2_ref[...], preferred_element_type=_f32)
             + bl2_ref[...])
        m = jnp.max(o, axis=1, keepdims=True)
        lse = m + jnp.log(jnp.sum(jnp.exp(o - m), axis=1, keepdims=True))
        out_ref[...] = o - lse


def _tc_pool(batch3, x1, x2, x3, wl1, bl1, wl2, bl2):
    grid = N_PAD // CH
    nclass = wl2.shape[1]
    return pl.pallas_call(
        _pool_body,
        grid=(grid,),
        in_specs=[
            pl.BlockSpec((1, 1, CH), lambda i: (i, 0, 0)),
            pl.BlockSpec((CH, H), lambda i: (i, 0)),
            pl.BlockSpec((CH, H), lambda i: (i, 0)),
            pl.BlockSpec((CH, H), lambda i: (i, 0)),
            pl.BlockSpec((3 * H, H), lambda i: (0, 0)),
            pl.BlockSpec((1, H), lambda i: (0, 0)),
            pl.BlockSpec((H, nclass), lambda i: (0, 0)),
            pl.BlockSpec((1, nclass), lambda i: (0, 0)),
        ],
        out_specs=pl.BlockSpec((NSEG, nclass), lambda i: (0, 0)),
        out_shape=jax.ShapeDtypeStruct((NSEG, nclass), _f32),
        scratch_shapes=[pltpu.VMEM((NSEG, 3 * H), _f32),
                        pltpu.VMEM((NSEG, CH), _f32)],
    )(batch3, x1, x2, x3, wl1, bl1.reshape(1, H), wl2,
      bl2.reshape(1, nclass))


def kernel(x, edge_index, batch, target_size, W1, b1, W2, b2, W3, b3,
           Wl1, bl1, Wl2, bl2):
    E = edge_index.shape[1]
    K = -(-E // (NC * NS * CH))
    K = -(-K // 4) * 4
    e_pad = NC * NS * K * CH - E

    src = jnp.concatenate(
        [edge_index[0], jnp.full((e_pad,), N_NODES, jnp.int32)])
    dst = jnp.concatenate(
        [edge_index[1], jnp.full((e_pad,), N_NODES, jnp.int32)])
    src_r = src.reshape(NC, NS, K, CH)
    dst_r = dst.reshape(NC, NS, K, CH)

    xpad = jnp.zeros((N_PAD, H), _f32).at[:N_NODES].set(x)
    zeros_h = jnp.zeros((N_PAD, H), _f32)
    zeros_d = jnp.zeros((N_PAD, DEGW), _f32)
    ones_d = jnp.ones((CH, DEGW), _f32)
    batch3 = jnp.concatenate(
        [batch.astype(jnp.int32),
         jnp.full((N_PAD - N_NODES,), NSEG, jnp.int32)]).reshape(
             N_PAD // CH, 1, CH)

    agg = _build_agg(K)
    deg = _build_deg(K)

    deg2 = deg(dst_r, ones_d, zeros_d)
    y1 = _tc_prep(xpad, W1, deg2)
    acc1 = agg(y1, src_r, dst_r, zeros_h)
    x1, y2 = _tc_mid(acc1, y1, deg2, b1, W2)
    acc2 = agg(y2, src_r, dst_r, zeros_h)
    x2, y3 = _tc_mid(acc2, y2, deg2, b2, W3)
    acc3 = agg(y3, src_r, dst_r, zeros_h)
    x3 = _tc_mid(acc3, y3, deg2, b3, None)
    return _tc_pool(batch3, x1, x2, x3, Wl1, bl1, Wl2, bl2)


# speedup vs baseline: 1.8834x; 1.8834x over previous
import functools

import jax
import jax.numpy as jnp
from jax import lax
from jax.experimental import pallas as pl
from jax.experimental.pallas import tpu as pltpu
from jax.experimental.pallas import tpu_sc as plsc

N_NODES = 10000
N_PAD = 10240
H = 128
NSEG = 64
NC = 2
NS = 16
CH = 128
RPS = N_PAD // NS
BR = 1280
DEGW = 128

_f32 = jnp.float32
FRAC0 = 0.65


def _sc_mesh():
    return plsc.VectorSubcoreMesh(core_axis_name="c", subcore_axis_name="s")


@functools.cache
def _build_agg(K, kf, ks):
    @functools.partial(
        pl.kernel,
        mesh=_sc_mesh(),
        out_type=jax.ShapeDtypeStruct((NC, N_PAD, H), _f32),
        scratch_types=[
            pltpu.VMEM((K, CH), jnp.int32),
            pltpu.VMEM((K, CH), jnp.int32),
            pltpu.VMEM((CH, H), _f32),
            pltpu.VMEM_SHARED((N_PAD, H), _f32),
            pltpu.SemaphoreType.DMA,
        ],
    )
    def agg(y_hbm, src_hbm, dst_hbm, z_hbm, out_hbm, src_v, dst_v, rows_v,
            acc_sh, sem):
        c = lax.axis_index("c")
        s = lax.axis_index("s")
        pltpu.sync_copy(src_hbm.at[c, s], src_v)
        pltpu.sync_copy(dst_hbm.at[c, s], dst_v)
        pltpu.sync_copy(z_hbm.at[pl.ds(s * RPS, RPS)],
                        acc_sh.at[pl.ds(s * RPS, RPS)])
        plsc.subcore_barrier()

        def body(j, carry):
            pltpu.async_copy(y_hbm.at[src_v.at[j]], rows_v, sem).wait()
            pltpu.sync_copy(rows_v, acc_sh.at[dst_v.at[j]], add=True)
            return carry

        kc = jnp.where(c == 0, kf, ks)
        lax.fori_loop(0, kc, body, 0)
        plsc.subcore_barrier()
        pltpu.sync_copy(acc_sh.at[pl.ds(s * RPS, RPS)],
                        out_hbm.at[c, pl.ds(s * RPS, RPS)])

    return agg


@functools.cache
def _build_deg(K, kf, ks, degw=DEGW):
    @functools.partial(
        pl.kernel,
        mesh=_sc_mesh(),
        out_type=jax.ShapeDtypeStruct((NC, N_PAD, degw), _f32),
        scratch_types=[
            pltpu.VMEM((K, CH), jnp.int32),
            pltpu.VMEM((CH, degw), _f32),
            pltpu.VMEM_SHARED((N_PAD, degw), _f32),
        ],
    )
    def deg(dst_hbm, ones_hbm, z_hbm, out_hbm, dst_v, ones_v, deg_sh):
        c = lax.axis_index("c")
        s = lax.axis_index("s")
        pltpu.sync_copy(dst_hbm.at[c, s], dst_v)
        pltpu.sync_copy(ones_hbm, ones_v)
        pltpu.sync_copy(z_hbm.at[pl.ds(s * RPS, RPS)],
                        deg_sh.at[pl.ds(s * RPS, RPS)])
        plsc.subcore_barrier()

        def body(j, carry):
            pltpu.sync_copy(ones_v, deg_sh.at[dst_v.at[j]], add=True)
            return carry

        kc = jnp.where(c == 0, kf, ks)
        lax.fori_loop(0, kc, body, 0)
        plsc.subcore_barrier()
        pltpu.sync_copy(deg_sh.at[pl.ds(s * RPS, RPS)],
                        out_hbm.at[c, pl.ds(s * RPS, RPS)])

    return deg


def _dinv_of(deg_blk):
    return lax.rsqrt(deg_blk[0, :, 0:1] + deg_blk[1, :, 0:1] + 1.0)


def _prep_body(x_ref, w_ref, deg_ref, y_ref):
    dinv = _dinv_of(deg_ref[...])
    y_ref[...] = jnp.dot(x_ref[...], w_ref[...],
                         preferred_element_type=_f32) * dinv


def _tc_prep(xpad, w, deg2):
    grid = N_PAD // BR
    return pl.pallas_call(
        _prep_body,
        grid=(grid,),
        in_specs=[
            pl.BlockSpec((BR, H), lambda i: (i, 0)),
            pl.BlockSpec((H, H), lambda i: (0, 0)),
            pl.BlockSpec((NC, BR, DEGW), lambda i: (0, i, 0)),
        ],
        out_specs=pl.BlockSpec((BR, H), lambda i: (i, 0)),
        out_shape=jax.ShapeDtypeStruct((N_PAD, H), _f32),
    )(xpad, w, deg2)


def _mid_body(acc_ref, y_ref, deg_ref, b_ref, w_ref, x_ref, ynext_ref):
    dinv = _dinv_of(deg_ref[...])
    acc = acc_ref[0] + acc_ref[1] + y_ref[...]
    xk = jnp.maximum(dinv * acc + b_ref[...], 0.0)
    row = pl.program_id(0) * BR + lax.broadcasted_iota(jnp.int32, (BR, 1), 0)
    xk = jnp.where(row < N_NODES, xk, 0.0)
    x_ref[...] = xk
    ynext_ref[...] = jnp.dot(xk, w_ref[...],
                             preferred_element_type=_f32) * dinv


def _mid_last_body(acc_ref, y_ref, deg_ref, b_ref, x_ref):
    dinv = _dinv_of(deg_ref[...])
    acc = acc_ref[0] + acc_ref[1] + y_ref[...]
    xk = jnp.maximum(dinv * acc + b_ref[...], 0.0)
    row = pl.program_id(0) * BR + lax.broadcasted_iota(jnp.int32, (BR, 1), 0)
    x_ref[...] = jnp.where(row < N_NODES, xk, 0.0)


def _tc_mid(acc2, y, deg2, b, w_next):
    grid = N_PAD // BR
    specs = [
        pl.BlockSpec((NC, BR, H), lambda i: (0, i, 0)),
        pl.BlockSpec((BR, H), lambda i: (i, 0)),
        pl.BlockSpec((NC, BR, DEGW), lambda i: (0, i, 0)),
        pl.BlockSpec((1, H), lambda i: (0, 0)),
    ]
    if w_next is None:
        return pl.pallas_call(
            _mid_last_body,
            grid=(grid,),
            in_specs=specs,
            out_specs=pl.BlockSpec((BR, H), lambda i: (i, 0)),
            out_shape=jax.ShapeDtypeStruct((N_PAD, H), _f32),
        )(acc2, y, deg2, b.reshape(1, H))
    return pl.pallas_call(
        _mid_body,
        grid=(grid,),
        in_specs=specs + [pl.BlockSpec((H, H), lambda i: (0, 0))],
        out_specs=[pl.BlockSpec((BR, H), lambda i: (i, 0)),
                   pl.BlockSpec((BR, H), lambda i: (i, 0))],
        out_shape=[jax.ShapeDtypeStruct((N_PAD, H), _f32),
                   jax.ShapeDtypeStruct((N_PAD, H), _f32)],
    )(acc2, y, deg2, b.reshape(1, H), w_next)


def _pool_body(b3_ref, x1_ref, x2_ref, x3_ref, wl1_ref, bl1_ref, wl2_ref,
               bl2_ref, out_ref, sums, cnts):
    i = pl.program_id(0)

    @pl.when(i == 0)
    def _init():
        sums[...] = jnp.zeros_like(sums)
        cnts[...] = jnp.zeros_like(cnts)

    bvals = b3_ref[...].reshape(1, CH)
    seg = lax.broadcasted_iota(jnp.int32, (NSEG, CH), 0)
    oh = (seg == bvals).astype(_f32)
    xj = jnp.concatenate([x1_ref[...], x2_ref[...], x3_ref[...]], axis=1)
    sums[...] += jnp.dot(oh, xj, preferred_element_type=_f32)
    cnts[...] += oh

    @pl.when(i == N_PAD // CH - 1)
    def _fin():
        cnt = jnp.sum(cnts[...], axis=1, keepdims=True)
        pooled = sums[...] / jnp.maximum(cnt, 1.0)
        h = jnp.maximum(
            jnp.dot(pooled, wl1_ref[...], preferred_element_type=_f32)
            + bl1_ref[...], 0.0)
        o = (jnp.dot(h, wl2_ref[...], preferred_element_type=_f32)
             + bl2_ref[...])
        m = jnp.max(o, axis=1, keepdims=True)
        lse = m + jnp.log(jnp.sum(jnp.exp(o - m), axis=1, keepdims=True))
        out_ref[...] = o - lse


def _tc_pool(batch3, x1, x2, x3, wl1, bl1, wl2, bl2):
    grid = N_PAD // CH
    nclass = wl2.shape[1]
    return pl.pallas_call(
        _pool_body,
        grid=(grid,),
        in_specs=[
            pl.BlockSpec((1, 1, CH), lambda i: (i, 0, 0)),
            pl.BlockSpec((CH, H), lambda i: (i, 0)),
            pl.BlockSpec((CH, H), lambda i: (i, 0)),
            pl.BlockSpec((CH, H), lambda i: (i, 0)),
            pl.BlockSpec((3 * H, H), lambda i: (0, 0)),
            pl.BlockSpec((1, H), lambda i: (0, 0)),
            pl.BlockSpec((H, nclass), lambda i: (0, 0)),
            pl.BlockSpec((1, nclass), lambda i: (0, 0)),
        ],
        out_specs=pl.BlockSpec((NSEG, nclass), lambda i: (0, 0)),
        out_shape=jax.ShapeDtypeStruct((NSEG, nclass), _f32),
        scratch_shapes=[pltpu.VMEM((NSEG, 3 * H), _f32),
                        pltpu.VMEM((NSEG, CH), _f32)],
    )(batch3, x1, x2, x3, wl1, bl1.reshape(1, H), wl2,
      bl2.reshape(1, nclass))


def kernel(x, edge_index, batch, target_size, W1, b1, W2, b2, W3, b3,
           Wl1, bl1, Wl2, bl2):
    E = edge_index.shape[1]
    per = NS * CH
    kf = -(-int(FRAC0 * E) // per)
    ks = -(-(E - per * kf) // per)
    K = max(kf, ks)
    e_pad = per * (kf + ks) - E

    def _split(ids):
        ids = jnp.concatenate(
            [ids, jnp.full((e_pad,), N_NODES, jnp.int32)])
        p0 = jnp.pad(ids[:per * kf].reshape(NS, kf, CH),
                     ((0, 0), (0, K - kf), (0, 0)),
                     constant_values=N_NODES)
        p1 = jnp.pad(ids[per * kf:].reshape(NS, ks, CH),
                     ((0, 0), (0, K - ks), (0, 0)),
                     constant_values=N_NODES)
        return jnp.stack([p0, p1])

    src_r = _split(edge_index[0])
    dst_r = _split(edge_index[1])

    xpad = jnp.zeros((N_PAD, H), _f32).at[:N_NODES].set(x)
    zeros_h = jnp.zeros((N_PAD, H), _f32)
    zeros_d = jnp.zeros((N_PAD, DEGW), _f32)
    ones_d = jnp.ones((CH, DEGW), _f32)
    batch3 = jnp.concatenate(
        [batch.astype(jnp.int32),
         jnp.full((N_PAD - N_NODES,), NSEG, jnp.int32)]).reshape(
             N_PAD // CH, 1, CH)

    agg = _build_agg(K, kf, ks)
    deg = _build_deg(K, kf, ks)

    deg2 = deg(dst_r, ones_d, zeros_d)
    y1 = _tc_prep(xpad, W1, deg2)
    acc1 = agg(y1, src_r, dst_r, zeros_h)
    x1, y2 = _tc_mid(acc1, y1, deg2, b1, W2)
    acc2 = agg(y2, src_r, dst_r, zeros_h)
    x2, y3 = _tc_mid(acc2, y2, deg2, b2, W3)
    acc3 = agg(y3, src_r, dst_r, zeros_h)
    x3 = _tc_mid(acc3, y3, deg2, b3, None)
    return _tc_pool(batch3, x1, x2, x3, Wl1, bl1, Wl2, bl2)


# speedup vs baseline: 2.0594x; 1.0934x over previous
import functools

import jax
import jax.numpy as jnp
from jax import lax
from jax.experimental import pallas as pl
from jax.experimental.pallas import tpu as pltpu
from jax.experimental.pallas import tpu_sc as plsc

N_NODES = 10000
N_PAD = 10240
H = 128
NSEG = 64
NC = 2
NS = 16
CH = 128
RPS = N_PAD // NS
BR = 1280
DEGW = 128

_f32 = jnp.float32
FRAC0 = 0.605
FRACD = 0.536
DEGO = 16
SUB = 1280 // 128


def _sc_mesh():
    return plsc.VectorSubcoreMesh(core_axis_name="c", subcore_axis_name="s")


@functools.cache
def _build_agg(K, kf, ks):
    @functools.partial(
        pl.kernel,
        mesh=_sc_mesh(),
        out_type=jax.ShapeDtypeStruct((NC, N_PAD, H), _f32),
        scratch_types=[
            pltpu.VMEM((K, CH), jnp.int32),
            pltpu.VMEM((K, CH), jnp.int32),
            pltpu.VMEM((CH, H), _f32),
            pltpu.VMEM_SHARED((N_PAD, H), _f32),
            pltpu.SemaphoreType.DMA,
        ],
    )
    def agg(y_hbm, src_hbm, dst_hbm, z_hbm, out_hbm, src_v, dst_v, rows_v,
            acc_sh, sem):
        c = lax.axis_index("c")
        s = lax.axis_index("s")
        pltpu.sync_copy(src_hbm.at[c, s], src_v)
        pltpu.sync_copy(dst_hbm.at[c, s], dst_v)
        pltpu.sync_copy(z_hbm.at[pl.ds(s * RPS, RPS)],
                        acc_sh.at[pl.ds(s * RPS, RPS)])
        plsc.subcore_barrier()

        def body(j, carry):
            pltpu.async_copy(y_hbm.at[src_v.at[j]], rows_v, sem).wait()
            pltpu.sync_copy(rows_v, acc_sh.at[dst_v.at[j]], add=True)
            return carry

        kc = jnp.where(c == 0, kf, ks)
        lax.fori_loop(0, kc, body, 0)
        plsc.subcore_barrier()
        pltpu.sync_copy(acc_sh.at[pl.ds(s * RPS, RPS)],
                        out_hbm.at[c, pl.ds(s * RPS, RPS)])

    return agg


@functools.cache
def _build_deg(K, kf, ks):
    @functools.partial(
        pl.kernel,
        mesh=_sc_mesh(),
        out_type=jax.ShapeDtypeStruct((NC, N_PAD, DEGW), _f32),
        scratch_types=[
            pltpu.VMEM((K, CH), jnp.int32),
            pltpu.VMEM((CH, DEGW), _f32),
            pltpu.VMEM_SHARED((N_PAD, DEGW), _f32),
        ],
    )
    def deg(dst_hbm, ones_hbm, z_hbm, out_hbm, dst_v, ones_v, deg_sh):
        c = lax.axis_index("c")
        s = lax.axis_index("s")
        pltpu.sync_copy(dst_hbm.at[c, s], dst_v)
        pltpu.sync_copy(ones_hbm, ones_v)
        pltpu.sync_copy(z_hbm.at[pl.ds(s * RPS, RPS)],
                        deg_sh.at[pl.ds(s * RPS, RPS)])
        plsc.subcore_barrier()

        def body(j, carry):
            pltpu.sync_copy(ones_v, deg_sh.at[dst_v.at[j]], add=True)
            return carry

        kc = jnp.where(c == 0, kf, ks)
        lax.fori_loop(0, kc, body, 0)
        plsc.subcore_barrier()
        pltpu.sync_copy(deg_sh.at[pl.ds(s * RPS, RPS)],
                        out_hbm.at[c, pl.ds(s * RPS, RPS)])

    return deg


def _dinv_of(deg_blk):
    return lax.rsqrt(deg_blk[0, :, 0:1] + deg_blk[1, :, 0:1] + 1.0)


def _prep_body(x_ref, w_ref, deg_ref, y_ref):
    dinv = _dinv_of(deg_ref[...])
    y_ref[...] = jnp.dot(x_ref[...], w_ref[...],
                         preferred_element_type=_f32) * dinv


def _tc_prep(xpad, w, deg2):
    grid = N_PAD // BR
    return pl.pallas_call(
        _prep_body,
        grid=(grid,),
        in_specs=[
            pl.BlockSpec((BR, H), lambda i: (i, 0)),
            pl.BlockSpec((H, H), lambda i: (0, 0)),
            pl.BlockSpec((NC, BR, DEGO), lambda i: (0, i, 0)),
        ],
        out_specs=pl.BlockSpec((BR, H), lambda i: (i, 0)),
        out_shape=jax.ShapeDtypeStruct((N_PAD, H), _f32),
    )(xpad, w, deg2)


def _mid_body(acc_ref, y_ref, deg_ref, b_ref, w_ref, x_ref, ynext_ref):
    dinv = _dinv_of(deg_ref[...])
    acc = acc_ref[0] + acc_ref[1] + y_ref[...]
    xk = jnp.maximum(dinv * acc + b_ref[...], 0.0)
    row = pl.program_id(0) * BR + lax.broadcasted_iota(jnp.int32, (BR, 1), 0)
    xk = jnp.where(row < N_NODES, xk, 0.0)
    x_ref[...] = xk
    ynext_ref[...] = jnp.dot(xk, w_ref[...],
                             preferred_element_type=_f32) * dinv


def _mid_last_body(acc_ref, y_ref, deg_ref, b_ref, x_ref):
    dinv = _dinv_of(deg_ref[...])
    acc = acc_ref[0] + acc_ref[1] + y_ref[...]
    xk = jnp.maximum(dinv * acc + b_ref[...], 0.0)
    row = pl.program_id(0) * BR + lax.broadcasted_iota(jnp.int32, (BR, 1), 0)
    x_ref[...] = jnp.where(row < N_NODES, xk, 0.0)


def _tc_mid(acc2, y, deg2, b, w_next):
    grid = N_PAD // BR
    specs = [
        pl.BlockSpec((NC, BR, H), lambda i: (0, i, 0)),
        pl.BlockSpec((BR, H), lambda i: (i, 0)),
        pl.BlockSpec((NC, BR, DEGO), lambda i: (0, i, 0)),
        pl.BlockSpec((1, H), lambda i: (0, 0)),
    ]
    if w_next is None:
        return pl.pallas_call(
            _mid_last_body,
            grid=(grid,),
            in_specs=specs,
            out_specs=pl.BlockSpec((BR, H), lambda i: (i, 0)),
            out_shape=jax.ShapeDtypeStruct((N_PAD, H), _f32),
        )(acc2, y, deg2, b.reshape(1, H))
    return pl.pallas_call(
        _mid_body,
        grid=(grid,),
        in_specs=specs + [pl.BlockSpec((H, H), lambda i: (0, 0))],
        out_specs=[pl.BlockSpec((BR, H), lambda i: (i, 0)),
                   pl.BlockSpec((BR, H), lambda i: (i, 0))],
        out_shape=[jax.ShapeDtypeStruct((N_PAD, H), _f32),
                   jax.ShapeDtypeStruct((N_PAD, H), _f32)],
    )(acc2, y, deg2, b.reshape(1, H), w_next)


def _pool_body(b3_ref, x1_ref, x2_ref, x3_ref, wl1_ref, bl1_ref, wl2_ref,
               bl2_ref, out_ref, sums, cnts):
    i = pl.program_id(0)

    @pl.when(i == 0)
    def _init():
        sums[...] = jnp.zeros_like(sums)
        cnts[...] = jnp.zeros_like(cnts)

    bv = b3_ref[...]
    x1a, x2a, x3a = x1_ref[...], x2_ref[...], x3_ref[...]
    seg = lax.broadcasted_iota(jnp.int32, (NSEG, CH), 0)
    for j in range(SUB):
        oh = (seg == bv[j]).astype(_f32)
        sl = slice(j * CH, (j + 1) * CH)
        xj = jnp.concatenate([x1a[sl], x2a[sl], x3a[sl]], axis=1)
        sums[...] += jnp.dot(oh, xj, preferred_element_type=_f32)
        cnts[...] += oh

    @pl.when(i == N_PAD // BR - 1)
    def _fin():
        cnt = jnp.sum(cnts[...], axis=1, keepdims=True)
        pooled = sums[...] / jnp.maximum(cnt, 1.0)
        h = jnp.maximum(
            jnp.dot(pooled, wl1_ref[...], preferred_element_type=_f32)
            + bl1_ref[...], 0.0)
        o = (jnp.dot(h, wl2_ref[...], preferred_element_type=_f32)
             + bl2_ref[...])
        m = jnp.max(o, axis=1, keepdims=True)
        lse = m + jnp.log(jnp.sum(jnp.exp(o - m), axis=1, keepdims=True))
        out_ref[...] = o - lse


def _tc_pool(batch3, x1, x2, x3, wl1, bl1, wl2, bl2):
    grid = N_PAD // BR
    nclass = wl2.shape[1]
    return pl.pallas_call(
        _pool_body,
        grid=(grid,),
        in_specs=[
            pl.BlockSpec((SUB, 1, CH), lambda i: (i, 0, 0)),
            pl.BlockSpec((BR, H), lambda i: (i, 0)),
            pl.BlockSpec((BR, H), lambda i: (i, 0)),
            pl.BlockSpec((BR, H), lambda i: (i, 0)),
            pl.BlockSpec((3 * H, H), lambda i: (0, 0)),
            pl.BlockSpec((1, H), lambda i: (0, 0)),
            pl.BlockSpec((H, nclass), lambda i: (0, 0)),
            pl.BlockSpec((1, nclass), lambda i: (0, 0)),
        ],
        out_specs=pl.BlockSpec((NSEG, nclass), lambda i: (0, 0)),
        out_shape=jax.ShapeDtypeStruct((NSEG, nclass), _f32),
        scratch_shapes=[pltpu.VMEM((NSEG, 3 * H), _f32),
                        pltpu.VMEM((NSEG, CH), _f32)],
    )(batch3, x1, x2, x3, wl1, bl1.reshape(1, H), wl2,
      bl2.reshape(1, nclass))


def kernel(x, edge_index, batch, target_size, W1, b1, W2, b2, W3, b3,
           Wl1, bl1, Wl2, bl2):
    E = edge_index.shape[1]
    per = NS * CH

    def _plan(frac):
        kf = -(-int(frac * E) // per)
        ks = -(-(E - per * kf) // per)
        return max(kf, ks), kf, ks

    def _split(ids, plan):
        K, kf, ks = plan
        ids = jnp.concatenate(
            [ids, jnp.full((per * (kf + ks) - E,), N_NODES, jnp.int32)])
        p0 = jnp.pad(ids[:per * kf].reshape(NS, kf, CH),
                     ((0, 0), (0, K - kf), (0, 0)),
                     constant_values=N_NODES)
        p1 = jnp.pad(ids[per * kf:].reshape(NS, ks, CH),
                     ((0, 0), (0, K - ks), (0, 0)),
                     constant_values=N_NODES)
        return jnp.stack([p0, p1])

    plan_a = _plan(FRAC0)
    plan_d = _plan(FRACD)
    src_r = _split(edge_index[0], plan_a)
    dst_r = _split(edge_index[1], plan_a)
    dst_d = _split(edge_index[1], plan_d)

    xpad = jnp.zeros((N_PAD, H), _f32).at[:N_NODES].set(x)
    zeros_h = jnp.zeros((N_PAD, H), _f32)
    ones_d = jnp.ones((CH, DEGW), _f32)
    batch3 = jnp.concatenate(
        [batch.astype(jnp.int32),
         jnp.full((N_PAD - N_NODES,), NSEG, jnp.int32)]).reshape(
             N_PAD // CH, 1, CH)

    agg = _build_agg(*plan_a)
    deg = _build_deg(*plan_d)

    deg2 = deg(dst_d, ones_d, zeros_h)[:, :, :DEGO]
    y1 = _tc_prep(xpad, W1, deg2)
    acc1 = agg(y1, src_r, dst_r, zeros_h)
    x1, y2 = _tc_mid(acc1, y1, deg2, b1, W2)
    acc2 = agg(y2, src_r, dst_r, zeros_h)
    x2, y3 = _tc_mid(acc2, y2, deg2, b2, W3)
    acc3 = agg(y3, src_r, dst_r, zeros_h)
    x3 = _tc_mid(acc3, y3, deg2, b3, None)
    return _tc_pool(batch3, x1, x2, x3, Wl1, bl1, Wl2, bl2)
